# baseline (speedup 1.0000x reference)
import functools

import jax
import jax.numpy as jnp
from jax import lax
from jax.experimental import pallas as pl
from jax.experimental.pallas import tpu as pltpu
from jax.experimental.pallas import tpu_sc as plsc

_NS = 4
_K = 4
_BUCKETS = 1048576 // 4
_MASK = _BUCKETS - 1
_E = 16
_BOUNDARY = 2
_SIGN_BITS = 8
_WINDOWS = [2, 4, 8, 16]
_PRIME_POOL = [2654435761, 2246822519, 3266489917, 2028178513, 1220703125,
               1610612741, 805306457, 402653189, 2862933555, 3037000493,
               2971215073, 2147483647, 1000000007, 998244353, 777767777,
               122949829]
_SALTS = [0, 3735928559, 3405691582, 2343432205, 4277009102, 12648430,
          3131961357, 3735929054]
_COND_PRIMES = [2654435761, 2246822519, 3266489917, 2028178513, 1220703125,
                1610612741, 805306457, 402653189][:_SIGN_BITS]

_B, _T = 4, 2048
_CHUNK = 256
_HALO = 16


def _i32(v):
    v &= 0xFFFFFFFF
    return v - (1 << 32) if v >= (1 << 31) else v


_PK = [[_i32(_PRIME_POOL[i] ^ _SALTS[k % len(_SALTS)]) for i in range(16)]
       for k in range(_K)]
_CPM = [p & _MASK for p in _COND_PRIMES]

_mesh = plsc.VectorSubcoreMesh(core_axis_name="c", subcore_axis_name="s")


@functools.partial(
    pl.kernel,
    out_type=jax.ShapeDtypeStruct((_B, _T, 16 * _E), jnp.float32),
    mesh=_mesh,
    compiler_params=pltpu.CompilerParams(needs_layout_passes=False),
    scratch_types=[
        pltpu.VMEM((_CHUNK + _HALO,), jnp.int32),
        pltpu.VMEM((16, _CHUNK), jnp.int32),
        pltpu.VMEM((16384,), jnp.int32),
        pltpu.VMEM((8 * _CHUNK * _E,), jnp.float32),
        pltpu.VMEM((16384,), jnp.float32),
        pltpu.VMEM((128, 16 * _E), jnp.float32),
        pltpu.VMEM((_SIGN_BITS * 8, _E), jnp.float32),
        pltpu.SemaphoreType.DMA,
        pltpu.SemaphoreType.DMA,
        pltpu.SemaphoreType.DMA,
    ],
)
def _mhp_kernel(tok_hbm, wv_hbm, tab_hbm, out_hbm,
                tok_v, key_v, eidx_v, semb_v, gbuf_v, slab_v, wv_v,
                sem_g, sem_w, sem_o):
    wid = lax.axis_index("s") * 2 + lax.axis_index("c")
    b = wid >> 3
    t0 = (wid & 7) << 8

    tok_off = pl.multiple_of(b * jnp.int32(_T + _HALO) + t0, 8)
    pltpu.sync_copy(tok_hbm.at[pl.ds(tok_off, _CHUNK + _HALO)], tok_v)
    wv_desc = pltpu.async_copy(wv_hbm, wv_v, sem_w)

    ji = lax.iota(jnp.int32, 16)
    ji16 = ji * jnp.int32(16)
    jrow = ji * jnp.int32(256)

    def hash_body(g, carry):
        p0 = g * jnp.int32(16)
        sh = [tok_v[pl.ds(p0 + _HALO - 1 - i, 16)] for i in range(16)]
        for k in range(_K):
            h = sh[0] * _PK[k][0]
            nxt = 1
            for s in range(_NS):
                w = _WINDOWS[s]
                for i in range(nxt, w):
                    h = h ^ (sh[i] * _PK[k][i])
                nxt = w
                key_v[jnp.int32(s * _K + k), pl.ds(p0, 16)] = h & _MASK
        return carry

    lax.fori_loop(jnp.int32(0), jnp.int32(_CHUNK // 16), hash_body,
                  jnp.int32(0))

    for r in range(2):
        def sbody(g, carry, tb=r * 4):
            p0 = g * jnp.int32(16)
            for ti in range(4):
                t = tb + ti
                kk = key_v[jnp.int32(t), pl.ds(p0, 16)]
                pbase = ji16 + (p0 * jnp.int32(16) + jnp.int32(ti * 4096))
                for e in range(16):
                    plsc.store_scatter(
                        eidx_v, [pbase + jnp.int32(e)],
                        kk + jnp.int32((t * 16 + e) * _BUCKETS))
            return carry

        lax.fori_loop(jnp.int32(0), jnp.int32(_CHUNK // 16), sbody,
                      jnp.int32(0))
        sdescs = []
        for ti in range(4):
            for jh in range(2):
                sdescs.append(pltpu.async_copy(
                    tab_hbm.at[eidx_v.at[pl.ds(ti * 4096 + jh * 2048, 2048)]],
                    semb_v.at[pl.ds((r * 4 + ti) * 4096 + jh * 2048, 2048)],
                    sem_g))
        for dsc in sdescs:
            dsc.wait()
    wv_desc.wait()

    for gb in range(4):
        tokrows = [((gb * 4 + gi) * 16 + ji) * jnp.int32(16)
                   for gi in range(4)]

        def mm_body(d, accs, tokrows=tokrows):
            k_ = d >> jnp.int32(4)
            e_ = d & jnp.int32(15)
            ef = jnp.full((16,), e_, jnp.int32)
            wrow = jnp.full((16,), k_, jnp.int32)
            wvs = [plsc.load_gather(wv_v, [wrow + jnp.int32(j * 8), ef])
                   for j in range(_SIGN_BITS)]
            accs = list(accs)
            off = jnp.full((16,), k_ * jnp.int32(4096) + e_, jnp.int32)
            for gi in range(4):
                col = plsc.load_gather(semb_v, [tokrows[gi] + off])
                for j in range(_SIGN_BITS):
                    accs[gi * 8 + j] = accs[gi * 8 + j] + col * wvs[j]
            return tuple(accs)

        accs = lax.fori_loop(
            jnp.int32(0), jnp.int32(8 * _E), mm_body,
            tuple(jnp.zeros((16,), jnp.float32) for _ in range(32)))

        for gi in range(4):
            g = gb * 4 + gi
            ck = jnp.zeros((16,), jnp.int32)
            for j in range(_SIGN_BITS):
                bit = accs[gi * 8 + j] > 0.0
                ck = ck ^ jnp.where(bit, jnp.int32(_CPM[j]), jnp.int32(0))
            for t in range(8, 16):
                cur = key_v[jnp.int32(t), pl.ds(g * 16, 16)]
                key_v[jnp.int32(t), pl.ds(g * 16, 16)] = cur ^ ck

    odescs = [None, None]
    for r in range(4):
        def fbody(i, carry, r4=jnp.int32(r * 4)):
            p0 = (r4 + i) * jnp.int32(16)
            base = i * jnp.int32(4096)
            for t in range(16):
                kk = key_v[jnp.int32(t), pl.ds(p0, 16)]
                pbase = jrow + (base + jnp.int32(t * 16))
                for e in range(16):
                    plsc.store_scatter(
                        eidx_v, [pbase + jnp.int32(e)],
                        kk + jnp.int32((t * 16 + e) * _BUCKETS))
            return carry

        lax.fori_loop(jnp.int32(0), jnp.int32(4), fbody, jnp.int32(0))
        q = r & 1
        gdescs = []
        for c in range(4):
            for jh in range(2):
                off = c * 4096 + jh * 2048
                gdescs.append(pltpu.async_copy(
                    tab_hbm.at[eidx_v.at[pl.ds(off, 2048)]],
                    gbuf_v.at[pl.ds(off, 2048)],
                    sem_g))
        for dsc in gdescs:
            dsc.wait()
        if odescs[q] is not None:
            odescs[q].wait()

        def rsh_body(i, carry, qrow=jnp.int32(q * 64)):
            row = qrow + (i >> jnp.int32(4))
            colb = (i & jnp.int32(15)) * jnp.int32(16)
            fl = i * jnp.int32(16)
            slab_v[row, pl.ds(colb, 16)] = gbuf_v[pl.ds(fl, 16)]
            return carry

        lax.fori_loop(jnp.int32(0), jnp.int32(1024), rsh_body, jnp.int32(0))
        odescs[q] = pltpu.async_copy(
            slab_v.at[pl.ds(q * 64, 64)],
            out_hbm.at[b, pl.ds(pl.multiple_of(t0 + jnp.int32(r * 64), 64),
                                64)],
            sem_o)
    for dsc in odescs:
        if dsc is not None:
            dsc.wait()


def kernel(tokens, tables, cond_W):
    tok32 = tokens.astype(jnp.int32)
    tok_pad = jnp.pad(tok32, ((0, 0), (_HALO, 0))).reshape(-1)
    wv = cond_W.astype(jnp.float32).reshape(_SIGN_BITS * 8, _E)
    tab = tables.transpose(0, 2, 1).reshape(16 * _E * _BUCKETS)
    return _mhp_kernel(tok_pad, wv, tab)


# speedup vs baseline: 1.8846x; 1.8846x over previous
import functools

import jax
import jax.numpy as jnp
from jax import lax
from jax.experimental import pallas as pl
from jax.experimental.pallas import tpu as pltpu
from jax.experimental.pallas import tpu_sc as plsc

_NS = 4
_K = 4
_BUCKETS = 1048576 // 4
_MASK = _BUCKETS - 1
_E = 16
_BOUNDARY = 2
_SIGN_BITS = 8
_WINDOWS = [2, 4, 8, 16]
_PRIME_POOL = [2654435761, 2246822519, 3266489917, 2028178513, 1220703125,
               1610612741, 805306457, 402653189, 2862933555, 3037000493,
               2971215073, 2147483647, 1000000007, 998244353, 777767777,
               122949829]
_SALTS = [0, 3735928559, 3405691582, 2343432205, 4277009102, 12648430,
          3131961357, 3735929054]
_COND_PRIMES = [2654435761, 2246822519, 3266489917, 2028178513, 1220703125,
                1610612741, 805306457, 402653189][:_SIGN_BITS]

_B, _T = 4, 2048
_CHUNK = 256
_HALO = 16


def _i32(v):
    v &= 0xFFFFFFFF
    return v - (1 << 32) if v >= (1 << 31) else v


_PK = [[_i32(_PRIME_POOL[i] ^ _SALTS[k % len(_SALTS)]) for i in range(16)]
       for k in range(_K)]
_CPM = [p & _MASK for p in _COND_PRIMES]

_mesh = plsc.VectorSubcoreMesh(core_axis_name="c", subcore_axis_name="s")


@functools.partial(
    pl.kernel,
    out_type=jax.ShapeDtypeStruct((_B, _T, 16 * _E), jnp.float32),
    mesh=_mesh,
    compiler_params=pltpu.CompilerParams(needs_layout_passes=False),
    scratch_types=[
        pltpu.VMEM((_CHUNK + _HALO,), jnp.int32),
        pltpu.VMEM((16, _CHUNK), jnp.int32),
        pltpu.VMEM((16384,), jnp.int32),
        pltpu.VMEM((8 * _CHUNK * _E,), jnp.float32),
        pltpu.VMEM((16384,), jnp.float32),
        pltpu.VMEM((128, 16 * _E), jnp.float32),
        pltpu.VMEM((_SIGN_BITS * 8, _E), jnp.float32),
        pltpu.SemaphoreType.DMA,
        pltpu.SemaphoreType.DMA,
        pltpu.SemaphoreType.DMA,
    ],
)
def _mhp_kernel(tok_hbm, wv_hbm, tab_hbm, out_hbm,
                tok_v, key_v, eidx_v, semb_v, gbuf_v, slab_v, wv_v,
                sem_g, sem_w, sem_o):
    wid = lax.axis_index("s") * 2 + lax.axis_index("c")
    b = wid >> 3
    t0 = (wid & 7) << 8

    tok_off = pl.multiple_of(b * jnp.int32(_T + _HALO) + t0, 8)
    pltpu.sync_copy(tok_hbm.at[pl.ds(tok_off, _CHUNK + _HALO)], tok_v)
    wv_desc = pltpu.async_copy(wv_hbm, wv_v, sem_w)

    ji = lax.iota(jnp.int32, 16)
    ji16 = ji * jnp.int32(16)
    jrow = ji * jnp.int32(256)

    def hash_body(g, carry):
        p0 = g * jnp.int32(16)
        sh = [tok_v[pl.ds(p0 + _HALO - 1 - i, 16)] for i in range(16)]
        for k in range(_K):
            h = sh[0] * _PK[k][0]
            nxt = 1
            for s in range(_NS):
                w = _WINDOWS[s]
                for i in range(nxt, w):
                    h = h ^ (sh[i] * _PK[k][i])
                nxt = w
                key_v[jnp.int32(s * _K + k), pl.ds(p0, 16)] = h & _MASK
        return carry

    lax.fori_loop(jnp.int32(0), jnp.int32(_CHUNK // 16), hash_body,
                  jnp.int32(0))

    for r in range(2):
        def sbody(g, carry, tb=r * 4):
            p0 = g * jnp.int32(16)
            for ti in range(4):
                t = tb + ti
                kk = key_v[jnp.int32(t), pl.ds(p0, 16)]
                khl = ((kk >> jnp.int32(7)) * jnp.int32(1024)) \
                    + (kk & jnp.int32(127))
                pbase = ji16 + (p0 * jnp.int32(16) + jnp.int32(ti * 4096))
                for e in range(16):
                    cte = (t * 2 + (e >> 3)) * 2097152 + (e & 7) * 128
                    plsc.store_scatter(eidx_v, [pbase + jnp.int32(e)],
                                       khl + jnp.int32(cte))
            return carry

        lax.fori_loop(jnp.int32(0), jnp.int32(_CHUNK // 16), sbody,
                      jnp.int32(0))
        sdescs = []
        for ti in range(4):
            for jh in range(2):
                sdescs.append(pltpu.async_copy(
                    tab_hbm.at[eidx_v.at[pl.ds(ti * 4096 + jh * 2048, 2048)]],
                    semb_v.at[pl.ds((r * 4 + ti) * 4096 + jh * 2048, 2048)],
                    sem_g))
        for dsc in sdescs:
            dsc.wait()
    wv_desc.wait()

    for gb in range(4):
        tokrows = [((gb * 4 + gi) * 16 + ji) * jnp.int32(16)
                   for gi in range(4)]

        def mm_body(d, accs, tokrows=tokrows):
            k_ = d >> jnp.int32(4)
            e_ = d & jnp.int32(15)
            ef = jnp.full((16,), e_, jnp.int32)
            wrow = jnp.full((16,), k_, jnp.int32)
            wvs = [plsc.load_gather(wv_v, [wrow + jnp.int32(j * 8), ef])
                   for j in range(_SIGN_BITS)]
            accs = list(accs)
            off = jnp.full((16,), k_ * jnp.int32(4096) + e_, jnp.int32)
            for gi in range(4):
                col = plsc.load_gather(semb_v, [tokrows[gi] + off])
                for j in range(_SIGN_BITS):
                    accs[gi * 8 + j] = accs[gi * 8 + j] + col * wvs[j]
            return tuple(accs)

        accs = lax.fori_loop(
            jnp.int32(0), jnp.int32(8 * _E), mm_body,
            tuple(jnp.zeros((16,), jnp.float32) for _ in range(32)))

        for gi in range(4):
            g = gb * 4 + gi
            ck = jnp.zeros((16,), jnp.int32)
            for j in range(_SIGN_BITS):
                bit = accs[gi * 8 + j] > 0.0
                ck = ck ^ jnp.where(bit, jnp.int32(_CPM[j]), jnp.int32(0))
            for t in range(8, 16):
                cur = key_v[jnp.int32(t), pl.ds(g * 16, 16)]
                key_v[jnp.int32(t), pl.ds(g * 16, 16)] = cur ^ ck

    odescs = [None, None]
    for r in range(4):
        def fbody(i, carry, r4=jnp.int32(r * 4)):
            p0 = (r4 + i) * jnp.int32(16)
            base = i * jnp.int32(4096)
            for t in range(16):
                kk = key_v[jnp.int32(t), pl.ds(p0, 16)]
                khl = ((kk >> jnp.int32(7)) * jnp.int32(1024)) \
                    + (kk & jnp.int32(127))
                pbase = jrow + (base + jnp.int32(t * 16))
                for e in range(16):
                    cte = (t * 2 + (e >> 3)) * 2097152 + (e & 7) * 128
                    plsc.store_scatter(eidx_v, [pbase + jnp.int32(e)],
                                       khl + jnp.int32(cte))
            return carry

        lax.fori_loop(jnp.int32(0), jnp.int32(4), fbody, jnp.int32(0))
        q = r & 1
        gdescs = []
        for c in range(4):
            for jh in range(2):
                off = c * 4096 + jh * 2048
                gdescs.append(pltpu.async_copy(
                    tab_hbm.at[eidx_v.at[pl.ds(off, 2048)]],
                    gbuf_v.at[pl.ds(off, 2048)],
                    sem_g))
        for dsc in gdescs:
            dsc.wait()
        if odescs[q] is not None:
            odescs[q].wait()

        def rsh_body(i, carry, qrow=jnp.int32(q * 64)):
            row = qrow + (i >> jnp.int32(4))
            colb = (i & jnp.int32(15)) * jnp.int32(16)
            fl = i * jnp.int32(16)
            slab_v[row, pl.ds(colb, 16)] = gbuf_v[pl.ds(fl, 16)]
            return carry

        lax.fori_loop(jnp.int32(0), jnp.int32(1024), rsh_body, jnp.int32(0))
        odescs[q] = pltpu.async_copy(
            slab_v.at[pl.ds(q * 64, 64)],
            out_hbm.at[b, pl.ds(pl.multiple_of(t0 + jnp.int32(r * 64), 64),
                                64)],
            sem_o)
    for dsc in odescs:
        if dsc is not None:
            dsc.wait()


def kernel(tokens, tables, cond_W):
    tok32 = tokens.astype(jnp.int32)
    tok_pad = jnp.pad(tok32, ((0, 0), (_HALO, 0))).reshape(-1)
    wv = cond_W.astype(jnp.float32).reshape(_SIGN_BITS * 8, _E)
    tab = (tables.transpose(0, 2, 1)
           .reshape(16, 2, 8, 2048, 128)
           .transpose(0, 1, 3, 2, 4)
           .reshape(16 * _E * _BUCKETS))
    return _mhp_kernel(tok_pad, wv, tab)


# speedup vs baseline: 2.5575x; 1.3570x over previous
import functools

import jax
import jax.numpy as jnp
from jax import lax
from jax.experimental import pallas as pl
from jax.experimental.pallas import tpu as pltpu
from jax.experimental.pallas import tpu_sc as plsc

_NS = 4
_K = 4
_BUCKETS = 1048576 // 4
_MASK = _BUCKETS - 1
_E = 16
_BOUNDARY = 2
_SIGN_BITS = 8
_WINDOWS = [2, 4, 8, 16]
_PRIME_POOL = [2654435761, 2246822519, 3266489917, 2028178513, 1220703125,
               1610612741, 805306457, 402653189, 2862933555, 3037000493,
               2971215073, 2147483647, 1000000007, 998244353, 777767777,
               122949829]
_SALTS = [0, 3735928559, 3405691582, 2343432205, 4277009102, 12648430,
          3131961357, 3735929054]
_COND_PRIMES = [2654435761, 2246822519, 3266489917, 2028178513, 1220703125,
                1610612741, 805306457, 402653189][:_SIGN_BITS]

_B, _T = 4, 2048
_CHUNK = 256
_HALO = 16


def _i32(v):
    v &= 0xFFFFFFFF
    return v - (1 << 32) if v >= (1 << 31) else v


_PK = [[_i32(_PRIME_POOL[i] ^ _SALTS[k % len(_SALTS)]) for i in range(16)]
       for k in range(_K)]
_CPM = [p & _MASK for p in _COND_PRIMES]

_mesh = plsc.VectorSubcoreMesh(core_axis_name="c", subcore_axis_name="s")


@functools.partial(
    pl.kernel,
    out_type=jax.ShapeDtypeStruct((_B, _T, 16 * _E), jnp.float32),
    mesh=_mesh,
    compiler_params=pltpu.CompilerParams(needs_layout_passes=False),
    scratch_types=[
        pltpu.VMEM((_CHUNK + _HALO,), jnp.int32),
        pltpu.VMEM((16, _CHUNK), jnp.int32),
        pltpu.VMEM((16384,), jnp.int32),
        pltpu.VMEM((8 * _CHUNK * _E,), jnp.float32),
        pltpu.VMEM((16384,), jnp.float32),
        pltpu.VMEM((128, 16 * _E), jnp.float32),
        pltpu.VMEM((_SIGN_BITS * 8, _E), jnp.float32),
        pltpu.SemaphoreType.DMA,
        pltpu.SemaphoreType.DMA,
        pltpu.SemaphoreType.DMA,
    ],
)
def _mhp_kernel(tok_hbm, wv_hbm, tab_hbm, out_hbm,
                tok_v, key_v, eidx_v, semb_v, gbuf_v, slab_v, wv_v,
                sem_g, sem_w, sem_o):
    wid = lax.axis_index("s") * 2 + lax.axis_index("c")
    b = wid >> 3
    t0 = (wid & 7) << 8

    tok_off = pl.multiple_of(b * jnp.int32(_T + _HALO) + t0, 8)
    pltpu.sync_copy(tok_hbm.at[pl.ds(tok_off, _CHUNK + _HALO)], tok_v)
    wv_desc = pltpu.async_copy(wv_hbm, wv_v, sem_w)

    ji = lax.iota(jnp.int32, 16)
    ji16 = ji * jnp.int32(16)
    jrow = ji * jnp.int32(256)

    def hash_body(g, carry):
        p0 = g * jnp.int32(16)
        sh = [tok_v[pl.ds(p0 + _HALO - 1 - i, 16)] for i in range(16)]
        for k in range(_K):
            h = sh[0] * _PK[k][0]
            nxt = 1
            for s in range(_NS):
                w = _WINDOWS[s]
                for i in range(nxt, w):
                    h = h ^ (sh[i] * _PK[k][i])
                nxt = w
                key_v[jnp.int32(s * _K + k), pl.ds(p0, 16)] = h & _MASK
        return carry

    lax.fori_loop(jnp.int32(0), jnp.int32(_CHUNK // 16), hash_body,
                  jnp.int32(0))

    for r in range(2):
        def sbody(g, carry, tb=r * 4):
            p0 = g * jnp.int32(16)
            for ti in range(4):
                t = tb + ti
                kk = key_v[jnp.int32(t), pl.ds(p0, 16)]
                khl = ((kk >> jnp.int32(7)) * jnp.int32(1024)) \
                    + (kk & jnp.int32(127))
                pbase = ji16 + (p0 * jnp.int32(16) + jnp.int32(ti * 4096))
                for e in range(16):
                    cte = (t * 2 + (e >> 3)) * 2097152 + (e & 7) * 128
                    plsc.store_scatter(eidx_v, [pbase + jnp.int32(e)],
                                       khl + jnp.int32(cte))
            return carry

        lax.fori_loop(jnp.int32(0), jnp.int32(_CHUNK // 16), sbody,
                      jnp.int32(0))
        sdescs = []
        for ti in range(4):
            for jh in range(2):
                sdescs.append(pltpu.async_copy(
                    tab_hbm.at[eidx_v.at[pl.ds(ti * 4096 + jh * 2048, 2048)]],
                    semb_v.at[pl.ds((r * 4 + ti) * 4096 + jh * 2048, 2048)],
                    sem_g))
        for dsc in sdescs:
            dsc.wait()
    wv_desc.wait()

    for gb in range(4):
        tokrows = [((gb * 4 + gi) * 16 + ji) * jnp.int32(16)
                   for gi in range(4)]

        def mm_body(d, accs, tokrows=tokrows):
            k_ = d >> jnp.int32(4)
            e_ = d & jnp.int32(15)
            ef = jnp.full((16,), e_, jnp.int32)
            wrow = jnp.full((16,), k_, jnp.int32)
            wvs = [plsc.load_gather(wv_v, [wrow + jnp.int32(j * 8), ef])
                   for j in range(_SIGN_BITS)]
            accs = list(accs)
            off = jnp.full((16,), k_ * jnp.int32(4096) + e_, jnp.int32)
            for gi in range(4):
                col = plsc.load_gather(semb_v, [tokrows[gi] + off])
                for j in range(_SIGN_BITS):
                    accs[gi * 8 + j] = accs[gi * 8 + j] + col * wvs[j]
            return tuple(accs)

        accs = lax.fori_loop(
            jnp.int32(0), jnp.int32(8 * _E), mm_body,
            tuple(jnp.zeros((16,), jnp.float32) for _ in range(32)))

        for gi in range(4):
            g = gb * 4 + gi
            ck = jnp.zeros((16,), jnp.int32)
            for j in range(_SIGN_BITS):
                bit = accs[gi * 8 + j] > 0.0
                ck = ck ^ jnp.where(bit, jnp.int32(_CPM[j]), jnp.int32(0))
            for t in range(8, 16):
                cur = key_v[jnp.int32(t), pl.ds(g * 16, 16)]
                key_v[jnp.int32(t), pl.ds(g * 16, 16)] = cur ^ ck

    odescs = [None, None]
    for r in range(4):
        def fbody(i, carry, r4=jnp.int32(r * 4)):
            p0 = (r4 + i) * jnp.int32(16)
            base = i * jnp.int32(2048)
            jrow2 = ji * jnp.int32(128)
            for t in range(8, 16):
                kk = key_v[jnp.int32(t), pl.ds(p0, 16)]
                khl = ((kk >> jnp.int32(7)) * jnp.int32(1024)) \
                    + (kk & jnp.int32(127))
                pbase = jrow2 + (base + jnp.int32((t - 8) * 16))
                for e in range(16):
                    cte = (t * 2 + (e >> 3)) * 2097152 + (e & 7) * 128
                    plsc.store_scatter(eidx_v, [pbase + jnp.int32(e)],
                                       khl + jnp.int32(cte))
            return carry

        lax.fori_loop(jnp.int32(0), jnp.int32(4), fbody, jnp.int32(0))
        q = r & 1
        gdescs = []
        for c in range(4):
            off = c * 2048
            gdescs.append(pltpu.async_copy(
                tab_hbm.at[eidx_v.at[pl.ds(off, 2048)]],
                gbuf_v.at[pl.ds(off, 2048)],
                sem_g))
        if odescs[q] is not None:
            odescs[q].wait()

        def sh_body(tk, carry, qrow=jnp.int32(q * 64), g0=jnp.int32(r * 64)):
            row = qrow + tk
            gt16 = (g0 + tk) * jnp.int32(16)
            for t in range(8):
                slab_v[row, pl.ds(t * 16, 16)] = \
                    semb_v[pl.ds(jnp.int32(t * 4096) + gt16, 16)]
            return carry

        lax.fori_loop(jnp.int32(0), jnp.int32(64), sh_body, jnp.int32(0))
        for dsc in gdescs:
            dsc.wait()

        def lg_body(tk, carry, qrow=jnp.int32(q * 64)):
            row = qrow + tk
            fl = tk * jnp.int32(128)
            for t in range(8):
                slab_v[row, pl.ds(128 + t * 16, 16)] = \
                    gbuf_v[pl.ds(fl + jnp.int32(t * 16), 16)]
            return carry

        lax.fori_loop(jnp.int32(0), jnp.int32(64), lg_body, jnp.int32(0))
        odescs[q] = pltpu.async_copy(
            slab_v.at[pl.ds(q * 64, 64)],
            out_hbm.at[b, pl.ds(pl.multiple_of(t0 + jnp.int32(r * 64), 64),
                                64)],
            sem_o)
    for dsc in odescs:
        if dsc is not None:
            dsc.wait()


def kernel(tokens, tables, cond_W):
    tok32 = tokens.astype(jnp.int32)
    tok_pad = jnp.pad(tok32, ((0, 0), (_HALO, 0))).reshape(-1)
    wv = cond_W.astype(jnp.float32).reshape(_SIGN_BITS * 8, _E)
    tab = (tables.transpose(0, 2, 1)
           .reshape(16, 2, 8, 2048, 128)
           .transpose(0, 1, 3, 2, 4)
           .reshape(16 * _E * _BUCKETS))
    return _mhp_kernel(tok_pad, wv, tab)
